# baseline (speedup 1.0000x reference)
import functools

import jax
import jax.numpy as jnp
from jax import lax
from jax.experimental import pallas as pl
from jax.experimental.pallas import tpu as pltpu
from jax.experimental.pallas import tpu_sc as plsc

_N = 10000
_NP = 10240
_F = 128
_B = 16
_U = 128
_NC = 2
_NS = 16
_KE = 128
_RING = 4
_RPT = _NP // _NS
_CPC = _F // _NC

_GDN = lax.GatherDimensionNumbers(
    offset_dims=(), collapsed_slice_dims=(0,), start_index_map=(0,))


def _lane_bcast(v16, lane):
    idx = jnp.full((16, 1), lane, jnp.int32)
    return lax.gather(v16, idx, _GDN, (1,),
                      mode=lax.GatherScatterMode.PROMISE_IN_BOUNDS)


def _sc_body(nblk, a0_hbm, rows_hbm, cols_hbm, vals_hbm, a1_hbm, a2_hbm,
             y1_sp, zeros_v, rows_v, cols_v, vals_v,
             gbuf, sbuf,
             gs0, gs1, gs2, gs3, ss0, ss1, ss2, ss3):
    gsems = (gs0, gs1, gs2, gs3)
    ssems = (ss0, ss1, ss2, ss3)
    cid = lax.axis_index("c")
    sid = lax.axis_index("s")

    pltpu.sync_copy(rows_hbm.at[sid], rows_v)
    pltpu.sync_copy(cols_hbm.at[sid], cols_v)
    pltpu.sync_copy(vals_hbm.at[sid], vals_v)

    def _fill_zero(i, _):
        zeros_v[i, :] = jnp.zeros((_B,), jnp.float32)
        return 0
    lax.fori_loop(0, _RPT, _fill_zero, 0)

    r0 = sid * _RPT

    def _phase(src_hbm, chunk, y_sp):
        def _fire_gather(j, slot):
            return pltpu.async_copy(src_hbm.at[chunk].at[cols_v.at[j]],
                                    gbuf.at[slot], gsems[slot])

        def _scale_block(j, slot):
            for g in range(_KE // 16):
                vals16 = vals_v[pl.ds(j * _KE + g * 16, 16)]
                for l in range(16):
                    e = g * 16 + l
                    sbuf[slot, e, :] = gbuf[slot, e, :] * _lane_bcast(vals16, l)

        for slot in range(_RING):
            _fire_gather(slot, slot)
        nlap = nblk // _RING

        def _lap(i, _):
            for slot in range(_RING):
                j = i * _RING + slot
                @pl.when(i > 0)
                def _():
                    pltpu.make_async_copy(sbuf.at[slot],
                                          y_sp.at[rows_v.at[j]],
                                          ssems[slot]).wait()
                pltpu.make_async_copy(src_hbm.at[chunk].at[cols_v.at[j]],
                                      gbuf.at[slot], gsems[slot]).wait()
                _scale_block(j, slot)
                pltpu.async_copy(sbuf.at[slot], y_sp.at[rows_v.at[j]],
                                 ssems[slot], add=True)

                @pl.when(j + _RING < nblk)
                def _():
                    _fire_gather(j + _RING, slot)
            return 0
        lax.fori_loop(0, nlap, _lap, 0)
        for slot in range(_RING):
            j = (nlap - 1) * _RING + slot
            pltpu.make_async_copy(sbuf.at[slot], y_sp.at[rows_v.at[j]],
                                  ssems[slot]).wait()

    def _pass_a(ci, _):
        chunk = cid * _CPC + ci
        pltpu.sync_copy(zeros_v, y1_sp.at[pl.ds(r0, _RPT)])
        plsc.subcore_barrier()
        _phase(a0_hbm, chunk, y1_sp)
        plsc.subcore_barrier()
        pltpu.sync_copy(y1_sp.at[pl.ds(r0, _RPT)],
                        a1_hbm.at[chunk].at[pl.ds(r0, _RPT)])
        return 0
    lax.fori_loop(0, _CPC, _pass_a, 0)

    def _pass_b(ci, _):
        chunk = cid * _CPC + ci
        pltpu.sync_copy(zeros_v, y1_sp.at[pl.ds(r0, _RPT)])
        plsc.subcore_barrier()
        _phase(a1_hbm, chunk, y1_sp)
        plsc.subcore_barrier()
        pltpu.sync_copy(y1_sp.at[pl.ds(r0, _RPT)],
                        a2_hbm.at[chunk].at[pl.ds(r0, _RPT)])
        return 0
    lax.fori_loop(0, _CPC, _pass_b, 0)


def _sc_chebyshev(a0, rows3, cols3, vals3):
    nblk = rows3.shape[1]
    mesh = plsc.VectorSubcoreMesh(core_axis_name="c", subcore_axis_name="s")
    f = pl.kernel(
        functools.partial(_sc_body, nblk),
        out_type=(jax.ShapeDtypeStruct((_F, _NP, _B), jnp.float32),
                  jax.ShapeDtypeStruct((_F, _NP, _B), jnp.float32)),
        mesh=mesh,
        scratch_types=[
            pltpu.VMEM_SHARED((_NP, _B), jnp.float32),
            pltpu.VMEM((_RPT, _B), jnp.float32),
            pltpu.VMEM((nblk, _KE), jnp.int32),
            pltpu.VMEM((nblk, _KE), jnp.int32),
            pltpu.VMEM((nblk * _KE,), jnp.float32),
            pltpu.VMEM((_RING, _KE, _B), jnp.float32),
            pltpu.VMEM((_RING, _KE, _B), jnp.float32),
        ] + [pltpu.SemaphoreType.DMA] * (2 * _RING),
        compiler_params=pltpu.CompilerParams(needs_layout_passes=False,
                                             use_tc_tiling_on_sc=False),
    )
    return f(a0, rows3, cols3, vals3)


_RB = 1280


def _tc_body(a0_ref, a1_ref, a2_ref, w_ref, b_ref, o_ref):
    dims = (((0,), (0,)), ((), ()))
    acc = lax.dot_general(a0_ref[...], w_ref[0], dims,
                          precision=lax.Precision.HIGHEST,
                          preferred_element_type=jnp.float32)
    acc = acc + lax.dot_general(a1_ref[...], w_ref[1], dims,
                                precision=lax.Precision.HIGHEST,
                                preferred_element_type=jnp.float32)
    acc = acc + lax.dot_general(a2_ref[...], w_ref[2], dims,
                                precision=lax.Precision.HIGHEST,
                                preferred_element_type=jnp.float32)
    acc = acc + b_ref[...]
    o_ref[...] = jnp.transpose(acc.reshape(_RB // _B, _B, _U), (1, 0, 2))


def _tc_project(a0f, a1f, a2f, wr, b2):
    grid = (_N * _B // _RB,)
    return pl.pallas_call(
        _tc_body,
        grid=grid,
        in_specs=[
            pl.BlockSpec((_F, _RB), lambda i: (0, i)),
            pl.BlockSpec((_F, _RB), lambda i: (0, i)),
            pl.BlockSpec((_F, _RB), lambda i: (0, i)),
            pl.BlockSpec((3, _F, _U), lambda i: (0, 0, 0)),
            pl.BlockSpec((1, _U), lambda i: (0, 0)),
        ],
        out_specs=pl.BlockSpec((_B, _RB // _B, _U), lambda i: (0, i, 0)),
        out_shape=jax.ShapeDtypeStruct((_B, _N, _U), jnp.float32),
    )(a0f, a1f, a2f, wr, b2)


def kernel(inputs, weights, biases, sup_rows, sup_cols, sup_vals):
    batch = inputs.shape[0]
    x = inputs.reshape(batch, _N, _F)
    a0 = jnp.transpose(x, (2, 1, 0))
    a0 = jnp.pad(a0, ((0, 0), (0, _NP - _N), (0, 0)))

    nnz = sup_rows.shape[0]
    nblk = -(-nnz // (_NS * _KE))
    nblk = -(-nblk // _RING) * _RING
    pad = nblk * _NS * _KE - nnz
    rows3 = jnp.concatenate(
        [sup_rows.astype(jnp.int32), jnp.zeros((pad,), jnp.int32)]
    ).reshape(_NS, nblk, _KE)
    cols3 = jnp.concatenate(
        [sup_cols.astype(jnp.int32), jnp.zeros((pad,), jnp.int32)]
    ).reshape(_NS, nblk, _KE)
    vals3 = jnp.concatenate(
        [sup_vals, jnp.zeros((pad,), jnp.float32)]
    ).reshape(_NS, nblk * _KE)

    a1, a2 = _sc_chebyshev(a0, rows3, cols3, vals3)

    wr = jnp.transpose(weights.reshape(_F, 3, _U), (1, 0, 2))
    wr = jnp.stack([wr[0] - wr[2], wr[1], 2.0 * wr[2]])
    b2 = biases.reshape(1, _U)
    out = _tc_project(a0.reshape(_F, _NP * _B), a1.reshape(_F, _NP * _B),
                      a2.reshape(_F, _NP * _B), wr, b2)
    return out


# speedup vs baseline: 1.2139x; 1.2139x over previous
import functools

import jax
import jax.numpy as jnp
from jax import lax
from jax.experimental import pallas as pl
from jax.experimental.pallas import tpu as pltpu
from jax.experimental.pallas import tpu_sc as plsc

_N = 10000
_NP = 10240
_F = 128
_B = 16
_U = 128
_NC = 2
_NS = 16
_KE = 128
_RING = 4
_RPT = _NP // _NS
_CPC = _F // _NC

_GDN = lax.GatherDimensionNumbers(
    offset_dims=(), collapsed_slice_dims=(0,), start_index_map=(0,))


def _lane_bcast(v16, lane):
    idx = jnp.full((16, 1), lane, jnp.int32)
    return lax.gather(v16, idx, _GDN, (1,),
                      mode=lax.GatherScatterMode.PROMISE_IN_BOUNDS)


def _sc_body(nblk, a0_hbm, rows_hbm, cols_hbm, vals_hbm, a1_hbm, a2_hbm,
             buf_a, buf_b, zeros_v, rows_v, cols_v, vals_v,
             gbuf, sbuf,
             gs0, gs1, gs2, gs3, ss0, ss1, ss2, ss3):
    gsems = (gs0, gs1, gs2, gs3)
    ssems = (ss0, ss1, ss2, ss3)
    cid = lax.axis_index("c")
    sid = lax.axis_index("s")

    pltpu.sync_copy(rows_hbm.at[sid], rows_v)
    pltpu.sync_copy(cols_hbm.at[sid], cols_v)
    pltpu.sync_copy(vals_hbm.at[sid], vals_v)

    def _fill_zero(i, _):
        zeros_v[i, :] = jnp.zeros((_B,), jnp.float32)
        return 0
    lax.fori_loop(0, _RPT, _fill_zero, 0)

    r0 = sid * _RPT

    def _phase(src, y_sp):
        def _fire_gather(j, slot):
            return pltpu.async_copy(src.at[cols_v.at[j]],
                                    gbuf.at[slot], gsems[slot])

        def _scale_block(j, slot):
            def _g16(g, _):
                vals16 = vals_v[pl.ds(j * _KE + g * 16, 16)]
                for l in range(16):
                    bc = _lane_bcast(vals16, l)
                    e = g * 16 + l
                    sbuf[slot, e, :] = gbuf[slot, e, :] * bc
                return 0
            lax.fori_loop(0, _KE // 16, _g16, 0)

        for slot in range(_RING):
            _fire_gather(slot, slot)
        nlap = nblk // _RING

        def _lap(i, _):
            for slot in range(_RING):
                j = i * _RING + slot
                @pl.when(i > 0)
                def _():
                    pltpu.make_async_copy(sbuf.at[slot],
                                          y_sp.at[rows_v.at[j]],
                                          ssems[slot]).wait()
                pltpu.make_async_copy(src.at[cols_v.at[j]],
                                      gbuf.at[slot], gsems[slot]).wait()
                _scale_block(j, slot)
                pltpu.async_copy(sbuf.at[slot], y_sp.at[rows_v.at[j]],
                                 ssems[slot], add=True)

                @pl.when(j + _RING < nblk)
                def _():
                    _fire_gather(j + _RING, slot)
            return 0
        lax.fori_loop(0, nlap, _lap, 0)
        for slot in range(_RING):
            j = (nlap - 1) * _RING + slot
            pltpu.make_async_copy(sbuf.at[slot], y_sp.at[rows_v.at[j]],
                                  ssems[slot]).wait()

    def _chunk_iter(ci, _):
        chunk = cid * _CPC + ci
        pltpu.sync_copy(zeros_v, buf_a.at[pl.ds(r0, _RPT)])
        pltpu.sync_copy(zeros_v, buf_b.at[pl.ds(r0, _RPT)])
        plsc.subcore_barrier()
        _phase(a0_hbm.at[chunk], buf_a)
        plsc.subcore_barrier()
        pltpu.sync_copy(buf_a.at[pl.ds(r0, _RPT)],
                        a1_hbm.at[chunk].at[pl.ds(r0, _RPT)])
        _phase(buf_a, buf_b)
        plsc.subcore_barrier()
        pltpu.sync_copy(buf_b.at[pl.ds(r0, _RPT)],
                        a2_hbm.at[chunk].at[pl.ds(r0, _RPT)])
        return 0
    lax.fori_loop(0, _CPC, _chunk_iter, 0)


def _sc_chebyshev(a0, rows3, cols3, vals3):
    nblk = rows3.shape[1]
    mesh = plsc.VectorSubcoreMesh(core_axis_name="c", subcore_axis_name="s")
    f = pl.kernel(
        functools.partial(_sc_body, nblk),
        out_type=(jax.ShapeDtypeStruct((_F, _NP, _B), jnp.float32),
                  jax.ShapeDtypeStruct((_F, _NP, _B), jnp.float32)),
        mesh=mesh,
        scratch_types=[
            pltpu.VMEM_SHARED((_NP, _B), jnp.float32),
            pltpu.VMEM_SHARED((_NP, _B), jnp.float32),
            pltpu.VMEM((_RPT, _B), jnp.float32),
            pltpu.VMEM((nblk, _KE), jnp.int32),
            pltpu.VMEM((nblk, _KE), jnp.int32),
            pltpu.VMEM((nblk * _KE,), jnp.float32),
            pltpu.VMEM((_RING, _KE, _B), jnp.float32),
            pltpu.VMEM((_RING, _KE, _B), jnp.float32),
        ] + [pltpu.SemaphoreType.DMA] * (2 * _RING),
        compiler_params=pltpu.CompilerParams(needs_layout_passes=False,
                                             use_tc_tiling_on_sc=False),
    )
    return f(a0, rows3, cols3, vals3)


_RB = 1280


def _tc_body(a0_ref, a1_ref, a2_ref, w_ref, b_ref, o_ref):
    dims = (((0,), (0,)), ((), ()))
    acc = lax.dot_general(a0_ref[...], w_ref[0], dims,
                          precision=lax.Precision.HIGHEST,
                          preferred_element_type=jnp.float32)
    acc = acc + lax.dot_general(a1_ref[...], w_ref[1], dims,
                                precision=lax.Precision.HIGHEST,
                                preferred_element_type=jnp.float32)
    acc = acc + lax.dot_general(a2_ref[...], w_ref[2], dims,
                                precision=lax.Precision.HIGHEST,
                                preferred_element_type=jnp.float32)
    acc = acc + b_ref[...]
    o_ref[...] = jnp.transpose(acc.reshape(_RB // _B, _B, _U), (1, 0, 2))


def _tc_project(a0f, a1f, a2f, wr, b2):
    grid = (_N * _B // _RB,)
    return pl.pallas_call(
        _tc_body,
        grid=grid,
        in_specs=[
            pl.BlockSpec((_F, _RB), lambda i: (0, i)),
            pl.BlockSpec((_F, _RB), lambda i: (0, i)),
            pl.BlockSpec((_F, _RB), lambda i: (0, i)),
            pl.BlockSpec((3, _F, _U), lambda i: (0, 0, 0)),
            pl.BlockSpec((1, _U), lambda i: (0, 0)),
        ],
        out_specs=pl.BlockSpec((_B, _RB // _B, _U), lambda i: (0, i, 0)),
        out_shape=jax.ShapeDtypeStruct((_B, _N, _U), jnp.float32),
    )(a0f, a1f, a2f, wr, b2)


def kernel(inputs, weights, biases, sup_rows, sup_cols, sup_vals):
    batch = inputs.shape[0]
    x = inputs.reshape(batch, _N, _F)
    a0 = jnp.transpose(x, (2, 1, 0))
    a0 = jnp.pad(a0, ((0, 0), (0, _NP - _N), (0, 0)))

    nnz = sup_rows.shape[0]
    nblk = -(-nnz // (_NS * _KE))
    nblk = -(-nblk // _RING) * _RING
    pad = nblk * _NS * _KE - nnz
    rows3 = jnp.concatenate(
        [sup_rows.astype(jnp.int32), jnp.zeros((pad,), jnp.int32)]
    ).reshape(_NS, nblk, _KE)
    cols3 = jnp.concatenate(
        [sup_cols.astype(jnp.int32), jnp.zeros((pad,), jnp.int32)]
    ).reshape(_NS, nblk, _KE)
    vals3 = jnp.concatenate(
        [sup_vals, jnp.zeros((pad,), jnp.float32)]
    ).reshape(_NS, nblk * _KE)

    a1, a2 = _sc_chebyshev(a0, rows3, cols3, vals3)

    wr = jnp.transpose(weights.reshape(_F, 3, _U), (1, 0, 2))
    wr = jnp.stack([wr[0] - wr[2], wr[1], 2.0 * wr[2]])
    b2 = biases.reshape(1, _U)
    out = _tc_project(a0.reshape(_F, _NP * _B), a1.reshape(_F, _NP * _B),
                      a2.reshape(_F, _NP * _B), wr, b2)
    return out


# speedup vs baseline: 1.3693x; 1.1280x over previous
import functools

import jax
import jax.numpy as jnp
from jax import lax
from jax.experimental import pallas as pl
from jax.experimental.pallas import tpu as pltpu
from jax.experimental.pallas import tpu_sc as plsc

_N = 10000
_NP = 10240
_F = 128
_B = 16
_U = 128
_NC = 2
_NS = 16
_W = 32
_NCH = (_F * _B) // _W
_CPC = _NCH // _NC
_KE = 128
_SB = 24
_NSB = 7
_RING = 4
_RPT = _NP // _NS

_GDN = lax.GatherDimensionNumbers(
    offset_dims=(), collapsed_slice_dims=(0,), start_index_map=(0,))


def _lane_bcast(v16, lane):
    idx = jnp.full((16, 1), lane, jnp.int32)
    return lax.gather(v16, idx, _GDN, (1,),
                      mode=lax.GatherScatterMode.PROMISE_IN_BOUNDS)


def _sc_body(a32_hbm, rows_hbm, cols_hbm, vals_hbm, a1_hbm, a2_hbm,
             buf_a, buf_b, zeros_v, rows_v, cols_v, vals_v,
             gbuf, sbuf,
             es0, es1, es2, gs0, gs1, gs2, gs3, ss0, ss1, ss2, ss3):
    esems = (es0, es1, es2)
    gsems = (gs0, gs1, gs2, gs3)
    ssems = (ss0, ss1, ss2, ss3)
    cid = lax.axis_index("c")
    sid = lax.axis_index("s")
    r0 = sid * _RPT

    def _fill_zero(i, _):
        zeros_v[i, :] = jnp.zeros((_B,), jnp.float32)
        return 0
    lax.fori_loop(0, _RPT, _fill_zero, 0)

    def _load_edges(sb, buf):
        pltpu.async_copy(rows_hbm.at[sid].at[sb], rows_v.at[buf], esems[buf])
        pltpu.async_copy(cols_hbm.at[sid].at[sb], cols_v.at[buf], esems[buf])
        pltpu.async_copy(vals_hbm.at[sid].at[sb], vals_v.at[buf], esems[buf])

    def _wait_edges(buf):
        pltpu.make_async_copy(rows_hbm.at[sid].at[0], rows_v.at[buf],
                              esems[buf]).wait()
        pltpu.make_async_copy(cols_hbm.at[sid].at[0], cols_v.at[buf],
                              esems[buf]).wait()
        pltpu.make_async_copy(vals_hbm.at[sid].at[0], vals_v.at[buf],
                              esems[buf]).wait()

    def _phase(src, y_sp):
        def _fire_gather(ebuf, jj, slot):
            pltpu.async_copy(src.at[cols_v.at[ebuf, jj]], gbuf.at[slot],
                             gsems[slot])

        def _wait_gather(slot):
            pltpu.make_async_copy(src.at[cols_v.at[0, 0]], gbuf.at[slot],
                                  gsems[slot]).wait()

        def _fire_scatter(ebuf, jj, slot):
            pltpu.async_copy(sbuf.at[slot], y_sp.at[rows_v.at[ebuf, jj]],
                             ssems[slot], add=True)

        def _wait_scatter(slot):
            pltpu.make_async_copy(sbuf.at[slot], y_sp.at[rows_v.at[0, 0]],
                                  ssems[slot]).wait()

        def _scale_block(ebuf, jj, slot):
            def _g16(g, _):
                vals16 = vals_v[ebuf, pl.ds(jj * _KE + g * 16, 16)]
                for l in range(16):
                    bc = _lane_bcast(vals16, l)
                    e = g * 16 + l
                    sbuf[slot, e, pl.ds(0, 16)] = \
                        gbuf[slot, e, pl.ds(0, 16)] * bc
                    sbuf[slot, e, pl.ds(16, 16)] = \
                        gbuf[slot, e, pl.ds(16, 16)] * bc
                return 0
            lax.fori_loop(0, _KE // 16, _g16, 0)

        _load_edges(0, 0)
        _load_edges(1, 1)
        _wait_edges(0)
        for slot in range(_RING):
            _fire_gather(0, slot, slot)

        for sb in range(_NSB):
            ebuf = sb % 3

            def _lap(i, _, sb=sb, ebuf=ebuf):
                if sb + 2 < _NSB:
                    @pl.when(i == 1)
                    def _():
                        _load_edges(sb + 2, (sb + 2) % 3)
                if sb + 1 < _NSB:
                    @pl.when(i == 4)
                    def _():
                        _wait_edges((sb + 1) % 3)
                for slot in range(_RING):
                    jj = i * _RING + slot
                    if sb == 0:
                        @pl.when(i > 0)
                        def _():
                            _wait_scatter(slot)
                    else:
                        _wait_scatter(slot)
                    _wait_gather(slot)
                    _scale_block(ebuf, jj, slot)
                    _fire_scatter(ebuf, jj, slot)
                    if sb + 1 < _NSB:
                        @pl.when(i < (_SB // _RING) - 1)
                        def _():
                            _fire_gather(ebuf, jj + _RING, slot)

                        @pl.when(i == (_SB // _RING) - 1)
                        def _():
                            _fire_gather((sb + 1) % 3, slot, slot)
                    else:
                        @pl.when(i < (_SB // _RING) - 1)
                        def _():
                            _fire_gather(ebuf, jj + _RING, slot)
                return 0
            lax.fori_loop(0, _SB // _RING, _lap, 0)
        for slot in range(_RING):
            _wait_scatter(slot)

    def _zero(buf):
        pltpu.sync_copy(zeros_v, buf.at[pl.ds(r0, _RPT), pl.ds(0, 16)])
        pltpu.sync_copy(zeros_v, buf.at[pl.ds(r0, _RPT), pl.ds(16, 16)])

    def _drain(buf, dst_hbm, chunk):
        for q in range(2):
            pltpu.sync_copy(buf.at[pl.ds(r0, _RPT), pl.ds(q * 16, 16)],
                            dst_hbm.at[2 * chunk + q].at[pl.ds(r0, _RPT)])

    def _chunk_iter(ci, _):
        chunk = cid * _CPC + ci
        _zero(buf_a)
        _zero(buf_b)
        plsc.subcore_barrier()
        _phase(a32_hbm.at[chunk], buf_a)
        plsc.subcore_barrier()
        _drain(buf_a, a1_hbm, chunk)
        _phase(buf_a, buf_b)
        plsc.subcore_barrier()
        _drain(buf_b, a2_hbm, chunk)
        return 0
    lax.fori_loop(0, _CPC, _chunk_iter, 0)


def _sc_chebyshev(a32, rows4, cols4, vals3):
    mesh = plsc.VectorSubcoreMesh(core_axis_name="c", subcore_axis_name="s")
    f = pl.kernel(
        _sc_body,
        out_type=(jax.ShapeDtypeStruct((_F, _NP, _B), jnp.float32),
                  jax.ShapeDtypeStruct((_F, _NP, _B), jnp.float32)),
        mesh=mesh,
        scratch_types=[
            pltpu.VMEM_SHARED((_NP, _W), jnp.float32),
            pltpu.VMEM_SHARED((_NP, _W), jnp.float32),
            pltpu.VMEM((_RPT, _B), jnp.float32),
            pltpu.VMEM((3, _SB, _KE), jnp.int32),
            pltpu.VMEM((3, _SB, _KE), jnp.int32),
            pltpu.VMEM((3, _SB * _KE), jnp.float32),
            pltpu.VMEM((_RING, _KE, _W), jnp.float32),
            pltpu.VMEM((_RING, _KE, _W), jnp.float32),
        ] + [pltpu.SemaphoreType.DMA] * (3 + 2 * _RING),
        compiler_params=pltpu.CompilerParams(needs_layout_passes=False,
                                             use_tc_tiling_on_sc=False),
    )
    return f(a32, rows4, cols4, vals3)


_RB = 1280


def _tc_body(x_ref, a1_ref, a2_ref, w_ref, b_ref, o_ref):
    dims = (((0,), (0,)), ((), ()))
    acc = lax.dot_general(a1_ref[...], w_ref[1], dims,
                          precision=lax.Precision.HIGHEST,
                          preferred_element_type=jnp.float32)
    acc = acc + lax.dot_general(a2_ref[...], w_ref[2], dims,
                                precision=lax.Precision.HIGHEST,
                                preferred_element_type=jnp.float32)
    acc3 = jnp.transpose(acc.reshape(_RB // _B, _B, _U), (1, 0, 2))
    acc0 = lax.dot_general(x_ref[...], w_ref[0], (((2,), (0,)), ((), ())),
                           precision=lax.Precision.HIGHEST,
                           preferred_element_type=jnp.float32)
    o_ref[...] = acc0 + acc3 + b_ref[...]


def _tc_project(x3, a1f, a2f, wr, b2):
    grid = (_N * _B // _RB,)
    return pl.pallas_call(
        _tc_body,
        grid=grid,
        in_specs=[
            pl.BlockSpec((_B, _RB // _B, _F), lambda i: (0, i, 0)),
            pl.BlockSpec((_F, _RB), lambda i: (0, i)),
            pl.BlockSpec((_F, _RB), lambda i: (0, i)),
            pl.BlockSpec((3, _F, _U), lambda i: (0, 0, 0)),
            pl.BlockSpec((1, _U), lambda i: (0, 0)),
        ],
        out_specs=pl.BlockSpec((_B, _RB // _B, _U), lambda i: (0, i, 0)),
        out_shape=jax.ShapeDtypeStruct((_B, _N, _U), jnp.float32),
    )(x3, a1f, a2f, wr, b2)


def kernel(inputs, weights, biases, sup_rows, sup_cols, sup_vals):
    batch = inputs.shape[0]
    x = inputs.reshape(batch, _N, _F)
    xp = jnp.pad(x, ((0, 0), (0, _NP - _N), (0, 0)))
    a32 = jnp.transpose(xp.reshape(batch, _NP, _NCH, 2),
                        (2, 1, 3, 0)).reshape(_NCH, _NP, _W)

    nnz = sup_rows.shape[0]
    per_tile = _NSB * _SB * _KE
    pad = _NS * per_tile - nnz
    rows4 = jnp.concatenate(
        [sup_rows.astype(jnp.int32), jnp.zeros((pad,), jnp.int32)]
    ).reshape(_NS, _NSB, _SB, _KE)
    cols4 = jnp.concatenate(
        [sup_cols.astype(jnp.int32), jnp.zeros((pad,), jnp.int32)]
    ).reshape(_NS, _NSB, _SB, _KE)
    vals3 = jnp.concatenate(
        [sup_vals, jnp.zeros((pad,), jnp.float32)]
    ).reshape(_NS, _NSB, _SB * _KE)

    a1, a2 = _sc_chebyshev(a32, rows4, cols4, vals3)

    wr = jnp.transpose(weights.reshape(_F, 3, _U), (1, 0, 2))
    wr = jnp.stack([wr[0] - wr[2], wr[1], 2.0 * wr[2]])
    b2 = biases.reshape(1, _U)
    out = _tc_project(x, a1.reshape(_F, _NP * _B), a2.reshape(_F, _NP * _B),
                      wr, b2)
    return out
